# baseline (speedup 1.0000x reference)
import math

import jax
import jax.numpy as jnp
from jax.experimental import pallas as pl
from jax.experimental.pallas import tpu as pltpu

_BN_EPS = 1e-5


def _round_up(n, m):
    return ((n + m - 1) // m) * m


def _mm_kernel(a_ref, b_ref, o_ref, acc_ref):
    @pl.when(pl.program_id(2) == 0)
    def _init():
        acc_ref[...] = jnp.zeros_like(acc_ref)

    acc_ref[...] += jnp.dot(a_ref[...], b_ref[...],
                            preferred_element_type=jnp.float32)

    @pl.when(pl.program_id(2) == pl.num_programs(2) - 1)
    def _store():
        o_ref[...] = acc_ref[...]


def _matmul(a, b, bm, bn, bk):
    m, kk = a.shape
    _, n = b.shape
    grid = (m // bm, n // bn, kk // bk)
    return pl.pallas_call(
        _mm_kernel,
        grid=grid,
        in_specs=[
            pl.BlockSpec((bm, bk), lambda i, j, k: (i, k)),
            pl.BlockSpec((bk, bn), lambda i, j, k: (k, j)),
        ],
        out_specs=pl.BlockSpec((bm, bn), lambda i, j, k: (i, j)),
        out_shape=jax.ShapeDtypeStruct((m, n), jnp.float32),
        scratch_shapes=[pltpu.VMEM((bm, bn), jnp.float32)],
    )(a, b)


def _bn(h):
    mu = h.mean(axis=0)
    var = h.var(axis=0)
    return (h - mu) * jax.lax.rsqrt(var + _BN_EPS)


def kernel(x, edge_index, W0, b0, pool_w, W1, b1, W2, b2):
    f32 = jnp.float32
    n, c_in = x.shape
    hid = W0.shape[1]
    row, col = edge_index[0], edge_index[1]
    k = int(math.ceil(0.5 * n))
    np_ = _round_up(n, 512)
    kp = _round_up(k, 512)

    deg = jnp.zeros((n,), f32).at[col].add(1.0) + 2.0
    dis = jax.lax.rsqrt(deg)
    enorm = dis[row] * dis[col]
    self_w = 2.0 * dis * dis

    x_pad = jnp.zeros((np_, c_in), f32).at[:n].set(x)
    h0 = _matmul(x_pad, W0, 512, 128, 128)
    agg0 = jnp.zeros((n, hid), f32).at[col].add(enorm[:, None] * h0[row])
    h = agg0 + self_w[:, None] * h0[:n] + b0
    h = _bn(jax.nn.relu(h))
    res0 = h

    score = jnp.tanh((h @ pool_w) / jnp.linalg.norm(pool_w))
    topv, perm = jax.lax.top_k(score, k)
    inv = jnp.full((n,), 2**30, jnp.int32).at[perm].set(
        jnp.arange(k, dtype=jnp.int32))

    bf16 = jnp.bfloat16
    we = jnp.where(row != col, 1.0, 0.0).astype(bf16)
    dk = jnp.arange(k)
    ar = jnp.zeros((kp, np_), bf16).at[inv[row], col].add(we, mode="drop")
    ar = ar.at[dk, perm].add(bf16(1.0))
    ac = jnp.zeros((np_, kp), bf16).at[row, inv[col]].add(we, mode="drop")
    ac = ac.at[perm, dk].add(bf16(1.0))
    ap = _matmul(ar, ac, 512, 512, 512)
    ahat = ap.at[dk, dk].set(2.0)
    degp = ahat.sum(axis=0)
    disp = jnp.where(degp > 0, jax.lax.rsqrt(degp), 0.0)

    hp_pad = jnp.zeros((kp, hid), f32).at[:k].set(h[perm] * topv[:, None])
    z = _matmul(hp_pad, W1, 512, 128, 128)
    z = disp[:, None] * z
    t = _matmul(z.T, ahat, 128, 512, 512).T
    outp = disp[:k, None] * t[:k] + b1
    hp2 = _bn(jax.nn.relu(outp))

    hu = res0.at[perm].add(hp2)

    hu_pad = jnp.zeros((np_, hid), f32).at[:n].set(hu)
    h2 = _matmul(hu_pad, W2, 512, 128, 128)
    agg2 = jnp.zeros((n, hid), f32).at[col].add(enorm[:, None] * h2[row])
    out = agg2 + self_w[:, None] * h2[:n] + b2
    return _bn(jax.nn.relu(out))


# speedup vs baseline: 1.2435x; 1.2435x over previous
import math

import jax
import jax.numpy as jnp
from jax.experimental import pallas as pl
from jax.experimental.pallas import tpu as pltpu

_BN_EPS = 1e-5


def _round_up(n, m):
    return ((n + m - 1) // m) * m


def _mm_kernel(a_ref, b_ref, o_ref, acc_ref):
    @pl.when(pl.program_id(2) == 0)
    def _init():
        acc_ref[...] = jnp.zeros_like(acc_ref)

    acc_ref[...] += jnp.dot(a_ref[...], b_ref[...],
                            preferred_element_type=jnp.float32)

    @pl.when(pl.program_id(2) == pl.num_programs(2) - 1)
    def _store():
        o_ref[...] = acc_ref[...]


def _matmul(a, b, bm, bn, bk):
    m, kk = a.shape
    _, n = b.shape
    grid = (m // bm, n // bn, kk // bk)
    return pl.pallas_call(
        _mm_kernel,
        grid=grid,
        in_specs=[
            pl.BlockSpec((bm, bk), lambda i, j, k: (i, k)),
            pl.BlockSpec((bk, bn), lambda i, j, k: (k, j)),
        ],
        out_specs=pl.BlockSpec((bm, bn), lambda i, j, k: (i, j)),
        out_shape=jax.ShapeDtypeStruct((m, n), jnp.float32),
        scratch_shapes=[pltpu.VMEM((bm, bn), jnp.float32)],
    )(a, b)


def _bn(h):
    mu = h.mean(axis=0)
    var = h.var(axis=0)
    return (h - mu) * jax.lax.rsqrt(var + _BN_EPS)


def kernel(x, edge_index, W0, b0, pool_w, W1, b1, W2, b2):
    f32 = jnp.float32
    n, c_in = x.shape
    hid = W0.shape[1]
    row, col = edge_index[0], edge_index[1]
    k = int(math.ceil(0.5 * n))
    np_ = _round_up(n, 512)
    kp = _round_up(k, 512)

    deg = jnp.zeros((n,), f32).at[col].add(1.0) + 2.0
    dis = jax.lax.rsqrt(deg)
    enorm = dis[row] * dis[col]
    self_w = 2.0 * dis * dis

    x_pad = jnp.zeros((np_, c_in), f32).at[:n].set(x)
    h0 = _matmul(x_pad, W0, 512, 128, 128)
    agg0 = jnp.zeros((n, hid), f32).at[col].add(enorm[:, None] * h0[row])
    h = agg0 + self_w[:, None] * h0[:n] + b0
    h = _bn(jax.nn.relu(h))
    res0 = h

    score = jnp.tanh((h @ pool_w) / jnp.linalg.norm(pool_w))
    topv, perm = jax.lax.top_k(score, k)
    inv = jnp.full((n,), 2**30, jnp.int32).at[perm].set(
        jnp.arange(k, dtype=jnp.int32))

    we = jnp.where(row != col, 1.0, 0.0).astype(f32)
    dk = jnp.arange(k)
    ar = jnp.zeros((kp, np_), f32).at[inv[row], col].add(we, mode="drop")
    ar = ar.at[dk, perm].add(1.0)
    ac = jnp.zeros((np_, kp), f32).at[row, inv[col]].add(we, mode="drop")
    ac = ac.at[perm, dk].add(1.0)
    ap = _matmul(ar.astype(jnp.bfloat16), ac.astype(jnp.bfloat16),
                 512, 512, 512)
    ahat = ap.at[dk, dk].set(2.0)
    degp = ahat.sum(axis=0)
    disp = jnp.where(degp > 0, jax.lax.rsqrt(degp), 0.0)

    hp_pad = jnp.zeros((kp, hid), f32).at[:k].set(h[perm] * topv[:, None])
    z = _matmul(hp_pad, W1, 512, 128, 128)
    z = disp[:, None] * z
    t = _matmul(z.T, ahat, 128, 512, 512).T
    outp = disp[:k, None] * t[:k] + b1
    hp2 = _bn(jax.nn.relu(outp))

    hu = res0.at[perm].add(hp2)

    hu_pad = jnp.zeros((np_, hid), f32).at[:n].set(hu)
    h2 = _matmul(hu_pad, W2, 512, 128, 128)
    agg2 = jnp.zeros((n, hid), f32).at[col].add(enorm[:, None] * h2[row])
    out = agg2 + self_w[:, None] * h2[:n] + b2
    return _bn(jax.nn.relu(out))
